# baseline (speedup 1.0000x reference)
import functools

import jax
import jax.numpy as jnp
from jax import lax
from jax.experimental import pallas as pl
from jax.experimental.pallas import tpu as pltpu
from jax.experimental.pallas import tpu_sc as plsc

NUM_CLASSES = 3
K = 5
M = 500
MP = 512
N = 65536
R = 16
FLAT = MP * K


def _topk_body(gt_ref, vp_ref, cd_ref, mask_ref, inds_ref, hmf_ref, rb_ref,
               aux_ref, dist_ref):
    f32 = jnp.float32
    i = pl.program_id(0)
    g = gt_ref[...]
    x, y, z = g[:, 0:1], g[:, 1:2], g[:, 2:3]
    sx, sy, sz = g[:, 3:4], g[:, 4:5], g[:, 5:6]
    head, clsf = g[:, 6:7], g[:, 7:8]

    valid_b = ((sx > 0) & (sy > 0) & (sz > 0)
               & (x >= f32(-75.2)) & (y >= f32(-75.2))
               & (x < f32(75.2)) & (y < f32(75.2)))
    valid = valid_b.astype(f32)

    cx = jnp.clip((x - f32(-75.2)) / f32(0.1) / f32(4.0), f32(0.0), f32(375.5))
    cy = jnp.clip((y - f32(-75.2)) / f32(0.1) / f32(4.0), f32(0.0), f32(375.5))
    dxw = sx / f32(0.1) / f32(4.0)
    dyw = sy / f32(0.1) / f32(4.0)
    radius = jnp.sqrt((dxw / 2.0) ** 2 + (dyw / 2.0) ** 2)

    vxh = vp_ref[0:1, :]
    vyh = vp_ref[1:2, :]
    pk = vp_ref[2:3, :]
    d0 = jnp.abs(vxh - cx) + jnp.abs(vyh - cy)
    dist_ref[...] = d0

    CW = 512
    NT = N // CW
    iota = lax.broadcasted_iota(jnp.int32, (R, N), 1)
    lio = lax.broadcasted_iota(jnp.int32, (R, CW), 1)
    inf = f32(jnp.inf)
    vals, idxs = [], []
    idx = None
    for k in range(K):
        run_f = jnp.full((R, CW), jnp.inf, f32)
        run_t = jnp.zeros((R, CW), jnp.int32)
        for t in range(NT):
            sl = slice(t * CW, (t + 1) * CW)
            if k == 0:
                dt = d0[:, sl]
            else:
                dt = dist_ref[:, sl]
                dt = jnp.where(iota[:, sl] == idx, inf, dt)
                if k < K - 1:
                    dist_ref[:, sl] = dt
            c = dt < run_f
            run_f = jnp.where(c, dt, run_f)
            run_t = jnp.where(c, jnp.int32(t), run_t)
        m = jnp.min(run_f, axis=1, keepdims=True)
        idx = jnp.min(jnp.where(run_f == m, run_t * CW + lio, N),
                      axis=1, keepdims=True)
        vals.append(m)
        idxs.append(idx)
    valsm = jnp.concatenate(vals, axis=1)
    indsm = jnp.concatenate(idxs, axis=1)

    cd_ref[...] = valsm * valid

    rio = lax.broadcasted_iota(jnp.int32, (R, K), 0)
    grow = rio + i * R
    base_mask = (valsm <= radius).astype(f32)
    mask_ref[...] = jnp.where(grow == 0, f32(1.0), base_mask) * valid

    inds_ref[...] = indsm * valid.astype(jnp.int32)

    cls_id = jnp.clip(clsf - 1.0, 0.0, float(NUM_CLASSES - 1)).astype(jnp.int32)
    hmf_ref[...] = jnp.where(valid_b, cls_id * N + indsm, NUM_CLASSES * N)

    lx, ly, lz = jnp.log(sx), jnp.log(sy), jnp.log(sz)
    ch, sh = jnp.cos(head), jnp.sin(head)
    cols = []
    for k in range(K):
        cols.extend([z, lx, ly, lz, ch, sh])
    rb_ref[...] = jnp.concatenate(cols, axis=1) * valid
    aux_ref[...] = jnp.concatenate([cx, cy, valid, valid], axis=1)


def _run_topk(gt_pad, vp):
    out_shapes = [
        jax.ShapeDtypeStruct((MP, K), jnp.float32),
        jax.ShapeDtypeStruct((MP, K), jnp.float32),
        jax.ShapeDtypeStruct((MP, K), jnp.int32),
        jax.ShapeDtypeStruct((MP, K), jnp.int32),
        jax.ShapeDtypeStruct((MP, K * 6), jnp.float32),
        jax.ShapeDtypeStruct((MP, 4), jnp.float32),
    ]
    small = lambda w, d: pl.BlockSpec((R, w), lambda i: (i, 0))
    return pl.pallas_call(
        _topk_body,
        grid=(MP // R,),
        in_specs=[
            pl.BlockSpec((R, 8), lambda i: (i, 0)),
            pl.BlockSpec((8, N), lambda i: (0, 0)),
        ],
        out_specs=[
            small(K, jnp.float32),
            small(K, jnp.float32),
            small(K, jnp.int32),
            small(K, jnp.int32),
            small(K * 6, jnp.float32),
            small(4, jnp.float32),
        ],
        out_shape=out_shapes,
        scratch_shapes=[pltpu.VMEM((R, N), jnp.float32)],
    )(gt_pad, vp)


def _hm_body(hmf_hbm, zero_hbm, pkt_hbm, aux_hbm, bidx_hbm,
             out_hbm, offx_hbm, offy_hbm,
             idx_v, row_v, bidx_v, aux_v, ox_v, oy_v):
    cid = lax.axis_index("c")
    sid = lax.axis_index("s")

    @pl.when((cid == 0) & (sid < NUM_CLASSES))
    def _():
        pltpu.sync_copy(zero_hbm.at[sid], row_v)
        pltpu.sync_copy(hmf_hbm, idx_v)
        ones = jnp.full((16,), 1.0, jnp.float32)
        base = sid * N

        def body(j, carry):
            ids = idx_v[pl.ds(j * 16, 16)]
            msk = (ids >= base) & (ids < base + N)
            loc = jnp.where(msk, ids - base, 0)
            plsc.store_scatter(row_v, [loc], ones, mask=msk)
            return carry

        lax.fori_loop(0, FLAT // 16, body, 0)
        pltpu.sync_copy(row_v, out_hbm.at[sid])

    n_gather = 8
    per_w = FLAT // 16 // n_gather

    @pl.when((cid == 1) & (sid < n_gather))
    def _():
        pltpu.sync_copy(pkt_hbm, row_v)
        pltpu.sync_copy(hmf_hbm, idx_v)
        pltpu.sync_copy(bidx_hbm, bidx_v)
        pltpu.sync_copy(aux_hbm, aux_v)
        base_j = sid * per_w

        def body(j, carry):
            jj = base_j + j
            ids = idx_v[pl.ds(jj * 16, 16)]
            vidx = ids & (N - 1)
            pkv = plsc.load_gather(row_v, [vidx])
            pki = pkv.astype(jnp.int32)
            vx = (pki >> 9).astype(jnp.float32)
            vy = (pki & 511).astype(jnp.float32)
            b4 = bidx_v[pl.ds(jj * 16, 16)] * 4
            cxv = plsc.load_gather(aux_v, [b4])
            cyv = plsc.load_gather(aux_v, [b4 + 1])
            vld = plsc.load_gather(aux_v, [b4 + 2])
            ox_v[pl.ds(j * 16, 16)] = ((cxv - vx) - 0.5) * vld
            oy_v[pl.ds(j * 16, 16)] = ((cyv - vy) - 0.5) * vld
            return carry

        lax.fori_loop(0, per_w, body, 0)
        pltpu.sync_copy(ox_v, offx_hbm.at[pl.ds(base_j * 16, per_w * 16)])
        pltpu.sync_copy(oy_v, offy_hbm.at[pl.ds(base_j * 16, per_w * 16)])


@functools.cache
def _hm_scatter():
    mesh = plsc.VectorSubcoreMesh(core_axis_name="c", subcore_axis_name="s")
    return pl.kernel(
        _hm_body,
        out_type=[
            jax.ShapeDtypeStruct((NUM_CLASSES, N), jnp.float32),
            jax.ShapeDtypeStruct((FLAT,), jnp.float32),
            jax.ShapeDtypeStruct((FLAT,), jnp.float32),
        ],
        mesh=mesh,
        scratch_types=[
            pltpu.VMEM((FLAT,), jnp.int32),
            pltpu.VMEM((N,), jnp.float32),
            pltpu.VMEM((FLAT,), jnp.int32),
            pltpu.VMEM((MP * 4,), jnp.float32),
            pltpu.VMEM((FLAT // 8,), jnp.float32),
            pltpu.VMEM((FLAT // 8,), jnp.float32),
        ],
        compiler_params=pltpu.CompilerParams(needs_layout_passes=False),
    )


def kernel(gt_boxes, spatial_indices):
    vox = spatial_indices.astype(jnp.float32)
    vxh = vox[:, 0] + 0.5
    vyh = vox[:, 1] + 0.5
    pk = vox[:, 0] * 512.0 + vox[:, 1]
    zero_row = jnp.zeros((N,), jnp.float32)
    vp = jnp.stack([vxh, vyh, pk, zero_row, zero_row, zero_row, zero_row,
                    zero_row], axis=0)
    gt_pad = jnp.zeros((MP, 8), jnp.float32).at[:M].set(gt_boxes)

    cd, mask, inds, hmf, rb6, aux = _run_topk(gt_pad, vp)
    bidx = (jnp.arange(FLAT, dtype=jnp.int32) // K).astype(jnp.int32)
    heatmap, offx, offy = _hm_scatter()(
        hmf.reshape(FLAT), jnp.zeros((NUM_CLASSES, N), jnp.float32),
        pk, aux.reshape(MP * 4), bidx)
    off = jnp.stack([offx, offy], axis=-1).reshape(MP, K, 2)
    ret_boxes = jnp.concatenate([off, rb6.reshape(MP, K, 6)], axis=-1)[:M]
    return heatmap, ret_boxes, cd[:M], inds[:M], mask[:M]


# speedup vs baseline: 1.0609x; 1.0609x over previous
import functools

import jax
import jax.numpy as jnp
from jax import lax
from jax.experimental import pallas as pl
from jax.experimental.pallas import tpu as pltpu
from jax.experimental.pallas import tpu_sc as plsc

NUM_CLASSES = 3
K = 5
M = 500
MP = 512
N = 65536
R = 32
FLAT = MP * K


def _topk_body(gt_ref, vp_ref, cd_ref, mask_ref, inds_ref, hmf_ref, rb_ref,
               aux_ref, dist_ref):
    f32 = jnp.float32
    i = pl.program_id(0)
    g = gt_ref[...]
    x, y, z = g[:, 0:1], g[:, 1:2], g[:, 2:3]
    sx, sy, sz = g[:, 3:4], g[:, 4:5], g[:, 5:6]
    head, clsf = g[:, 6:7], g[:, 7:8]

    valid_b = ((sx > 0) & (sy > 0) & (sz > 0)
               & (x >= f32(-75.2)) & (y >= f32(-75.2))
               & (x < f32(75.2)) & (y < f32(75.2)))
    valid = valid_b.astype(f32)

    cx = jnp.clip((x - f32(-75.2)) / f32(0.1) / f32(4.0), f32(0.0), f32(375.5))
    cy = jnp.clip((y - f32(-75.2)) / f32(0.1) / f32(4.0), f32(0.0), f32(375.5))
    dxw = sx / f32(0.1) / f32(4.0)
    dyw = sy / f32(0.1) / f32(4.0)
    radius = jnp.sqrt((dxw / 2.0) ** 2 + (dyw / 2.0) ** 2)

    vxh = vp_ref[0:1, :]
    vyh = vp_ref[1:2, :]
    pk = vp_ref[2:3, :]
    d0 = jnp.abs(vxh - cx) + jnp.abs(vyh - cy)
    dist_ref[...] = d0

    CW = 512
    NT = N // CW
    iota = lax.broadcasted_iota(jnp.int32, (R, N), 1)
    lio = lax.broadcasted_iota(jnp.int32, (R, CW), 1)
    inf = f32(jnp.inf)
    vals, idxs = [], []
    idx = None
    for k in range(K):
        run_f = jnp.full((R, CW), jnp.inf, f32)
        run_t = jnp.zeros((R, CW), jnp.int32)
        for t in range(NT):
            sl = slice(t * CW, (t + 1) * CW)
            if k == 0:
                dt = d0[:, sl]
            else:
                dt = dist_ref[:, sl]
                dt = jnp.where(iota[:, sl] == idx, inf, dt)
                if k < K - 1:
                    dist_ref[:, sl] = dt
            c = dt < run_f
            run_f = jnp.where(c, dt, run_f)
            run_t = jnp.where(c, jnp.int32(t), run_t)
        m = jnp.min(run_f, axis=1, keepdims=True)
        idx = jnp.min(jnp.where(run_f == m, run_t * CW + lio, N),
                      axis=1, keepdims=True)
        vals.append(m)
        idxs.append(idx)
    valsm = jnp.concatenate(vals, axis=1)
    indsm = jnp.concatenate(idxs, axis=1)

    cd_ref[...] = valsm * valid

    rio = lax.broadcasted_iota(jnp.int32, (R, K), 0)
    grow = rio + i * R
    base_mask = (valsm <= radius).astype(f32)
    mask_ref[...] = jnp.where(grow == 0, f32(1.0), base_mask) * valid

    inds_ref[...] = indsm * valid.astype(jnp.int32)

    cls_id = jnp.clip(clsf - 1.0, 0.0, float(NUM_CLASSES - 1)).astype(jnp.int32)
    hmf_ref[...] = jnp.where(valid_b, cls_id * N + indsm, NUM_CLASSES * N)

    lx, ly, lz = jnp.log(sx), jnp.log(sy), jnp.log(sz)
    ch, sh = jnp.cos(head), jnp.sin(head)
    cols = []
    for k in range(K):
        cols.extend([z, lx, ly, lz, ch, sh])
    rb_ref[...] = jnp.concatenate(cols, axis=1) * valid
    aux_ref[...] = jnp.concatenate([cx, cy, valid, valid], axis=1)


def _run_topk(gt_pad, vp):
    out_shapes = [
        jax.ShapeDtypeStruct((MP, K), jnp.float32),
        jax.ShapeDtypeStruct((MP, K), jnp.float32),
        jax.ShapeDtypeStruct((MP, K), jnp.int32),
        jax.ShapeDtypeStruct((MP, K), jnp.int32),
        jax.ShapeDtypeStruct((MP, K * 6), jnp.float32),
        jax.ShapeDtypeStruct((MP, 4), jnp.float32),
    ]
    small = lambda w, d: pl.BlockSpec((R, w), lambda i: (i, 0))
    return pl.pallas_call(
        _topk_body,
        grid=(MP // R,),
        in_specs=[
            pl.BlockSpec((R, 8), lambda i: (i, 0)),
            pl.BlockSpec((8, N), lambda i: (0, 0)),
        ],
        out_specs=[
            small(K, jnp.float32),
            small(K, jnp.float32),
            small(K, jnp.int32),
            small(K, jnp.int32),
            small(K * 6, jnp.float32),
            small(4, jnp.float32),
        ],
        out_shape=out_shapes,
        scratch_shapes=[pltpu.VMEM((R, N), jnp.float32)],
    )(gt_pad, vp)


def _hm_body(hmf_hbm, zero_hbm, pkt_hbm, aux_hbm, bidx_hbm,
             out_hbm, offx_hbm, offy_hbm,
             idx_v, row_v, bidx_v, aux_v, ox_v, oy_v):
    cid = lax.axis_index("c")
    sid = lax.axis_index("s")

    @pl.when((cid == 0) & (sid < NUM_CLASSES))
    def _():
        pltpu.sync_copy(zero_hbm.at[sid], row_v)
        pltpu.sync_copy(hmf_hbm, idx_v)
        ones = jnp.full((16,), 1.0, jnp.float32)
        base = sid * N

        def body(j, carry):
            ids = idx_v[pl.ds(j * 16, 16)]
            msk = (ids >= base) & (ids < base + N)
            loc = jnp.where(msk, ids - base, 0)
            plsc.store_scatter(row_v, [loc], ones, mask=msk)
            return carry

        lax.fori_loop(0, FLAT // 16, body, 0)
        pltpu.sync_copy(row_v, out_hbm.at[sid])

    n_gather = 8
    per_w = FLAT // 16 // n_gather

    @pl.when((cid == 1) & (sid < n_gather))
    def _():
        pltpu.sync_copy(pkt_hbm, row_v)
        pltpu.sync_copy(hmf_hbm, idx_v)
        pltpu.sync_copy(bidx_hbm, bidx_v)
        pltpu.sync_copy(aux_hbm, aux_v)
        base_j = sid * per_w

        def body(j, carry):
            jj = base_j + j
            ids = idx_v[pl.ds(jj * 16, 16)]
            vidx = ids & (N - 1)
            pkv = plsc.load_gather(row_v, [vidx])
            pki = pkv.astype(jnp.int32)
            vx = (pki >> 9).astype(jnp.float32)
            vy = (pki & 511).astype(jnp.float32)
            b4 = bidx_v[pl.ds(jj * 16, 16)] * 4
            cxv = plsc.load_gather(aux_v, [b4])
            cyv = plsc.load_gather(aux_v, [b4 + 1])
            vld = plsc.load_gather(aux_v, [b4 + 2])
            ox_v[pl.ds(j * 16, 16)] = ((cxv - vx) - 0.5) * vld
            oy_v[pl.ds(j * 16, 16)] = ((cyv - vy) - 0.5) * vld
            return carry

        lax.fori_loop(0, per_w, body, 0)
        pltpu.sync_copy(ox_v, offx_hbm.at[pl.ds(base_j * 16, per_w * 16)])
        pltpu.sync_copy(oy_v, offy_hbm.at[pl.ds(base_j * 16, per_w * 16)])


@functools.cache
def _hm_scatter():
    mesh = plsc.VectorSubcoreMesh(core_axis_name="c", subcore_axis_name="s")
    return pl.kernel(
        _hm_body,
        out_type=[
            jax.ShapeDtypeStruct((NUM_CLASSES, N), jnp.float32),
            jax.ShapeDtypeStruct((FLAT,), jnp.float32),
            jax.ShapeDtypeStruct((FLAT,), jnp.float32),
        ],
        mesh=mesh,
        scratch_types=[
            pltpu.VMEM((FLAT,), jnp.int32),
            pltpu.VMEM((N,), jnp.float32),
            pltpu.VMEM((FLAT,), jnp.int32),
            pltpu.VMEM((MP * 4,), jnp.float32),
            pltpu.VMEM((FLAT // 8,), jnp.float32),
            pltpu.VMEM((FLAT // 8,), jnp.float32),
        ],
        compiler_params=pltpu.CompilerParams(needs_layout_passes=False),
    )


def kernel(gt_boxes, spatial_indices):
    vox = spatial_indices.astype(jnp.float32)
    vxh = vox[:, 0] + 0.5
    vyh = vox[:, 1] + 0.5
    pk = vox[:, 0] * 512.0 + vox[:, 1]
    zero_row = jnp.zeros((N,), jnp.float32)
    vp = jnp.stack([vxh, vyh, pk, zero_row, zero_row, zero_row, zero_row,
                    zero_row], axis=0)
    gt_pad = jnp.zeros((MP, 8), jnp.float32).at[:M].set(gt_boxes)

    cd, mask, inds, hmf, rb6, aux = _run_topk(gt_pad, vp)
    bidx = (jnp.arange(FLAT, dtype=jnp.int32) // K).astype(jnp.int32)
    heatmap, offx, offy = _hm_scatter()(
        hmf.reshape(FLAT), jnp.zeros((NUM_CLASSES, N), jnp.float32),
        pk, aux.reshape(MP * 4), bidx)
    off = jnp.stack([offx, offy], axis=-1).reshape(MP, K, 2)
    ret_boxes = jnp.concatenate([off, rb6.reshape(MP, K, 6)], axis=-1)[:M]
    return heatmap, ret_boxes, cd[:M], inds[:M], mask[:M]


# speedup vs baseline: 1.1019x; 1.0386x over previous
import functools

import jax
import jax.numpy as jnp
from jax import lax
from jax.experimental import pallas as pl
from jax.experimental.pallas import tpu as pltpu
from jax.experimental.pallas import tpu_sc as plsc

NUM_CLASSES = 3
K = 5
M = 500
MP = 512
N = 65536
R = 64
FLAT = MP * K


def _topk_body(gt_ref, vp_ref, cd_ref, mask_ref, inds_ref, hmf_ref, rb_ref,
               aux_ref, dist_ref):
    f32 = jnp.float32
    i = pl.program_id(0)
    g = gt_ref[...]
    x, y, z = g[:, 0:1], g[:, 1:2], g[:, 2:3]
    sx, sy, sz = g[:, 3:4], g[:, 4:5], g[:, 5:6]
    head, clsf = g[:, 6:7], g[:, 7:8]

    valid_b = ((sx > 0) & (sy > 0) & (sz > 0)
               & (x >= f32(-75.2)) & (y >= f32(-75.2))
               & (x < f32(75.2)) & (y < f32(75.2)))
    valid = valid_b.astype(f32)

    cx = jnp.clip((x - f32(-75.2)) / f32(0.1) / f32(4.0), f32(0.0), f32(375.5))
    cy = jnp.clip((y - f32(-75.2)) / f32(0.1) / f32(4.0), f32(0.0), f32(375.5))
    dxw = sx / f32(0.1) / f32(4.0)
    dyw = sy / f32(0.1) / f32(4.0)
    radius = jnp.sqrt((dxw / 2.0) ** 2 + (dyw / 2.0) ** 2)

    vxh = vp_ref[0:1, :]
    vyh = vp_ref[1:2, :]
    pk = vp_ref[2:3, :]
    d0 = jnp.abs(vxh - cx) + jnp.abs(vyh - cy)
    dist_ref[...] = d0

    CW = 512
    NT = N // CW
    iota = lax.broadcasted_iota(jnp.int32, (R, N), 1)
    lio = lax.broadcasted_iota(jnp.int32, (R, CW), 1)
    inf = f32(jnp.inf)
    vals, idxs = [], []
    idx = None
    for k in range(K):
        run_f = jnp.full((R, CW), jnp.inf, f32)
        run_t = jnp.zeros((R, CW), jnp.int32)
        for t in range(NT):
            sl = slice(t * CW, (t + 1) * CW)
            if k == 0:
                dt = d0[:, sl]
            else:
                dt = dist_ref[:, sl]
                dt = jnp.where(iota[:, sl] == idx, inf, dt)
                if k < K - 1:
                    dist_ref[:, sl] = dt
            c = dt < run_f
            run_f = jnp.where(c, dt, run_f)
            run_t = jnp.where(c, jnp.int32(t), run_t)
        m = jnp.min(run_f, axis=1, keepdims=True)
        idx = jnp.min(jnp.where(run_f == m, run_t * CW + lio, N),
                      axis=1, keepdims=True)
        vals.append(m)
        idxs.append(idx)
    valsm = jnp.concatenate(vals, axis=1)
    indsm = jnp.concatenate(idxs, axis=1)

    cd_ref[...] = valsm * valid

    rio = lax.broadcasted_iota(jnp.int32, (R, K), 0)
    grow = rio + i * R
    base_mask = (valsm <= radius).astype(f32)
    mask_ref[...] = jnp.where(grow == 0, f32(1.0), base_mask) * valid

    inds_ref[...] = indsm * valid.astype(jnp.int32)

    cls_id = jnp.clip(clsf - 1.0, 0.0, float(NUM_CLASSES - 1)).astype(jnp.int32)
    hmf_ref[...] = jnp.where(valid_b, cls_id * N + indsm, NUM_CLASSES * N)

    lx, ly, lz = jnp.log(sx), jnp.log(sy), jnp.log(sz)
    ch, sh = jnp.cos(head), jnp.sin(head)
    cols = []
    for k in range(K):
        cols.extend([z, lx, ly, lz, ch, sh])
    rb_ref[...] = jnp.concatenate(cols, axis=1) * valid
    aux_ref[...] = jnp.concatenate([cx, cy, valid, valid], axis=1)


def _run_topk(gt_pad, vp):
    out_shapes = [
        jax.ShapeDtypeStruct((MP, K), jnp.float32),
        jax.ShapeDtypeStruct((MP, K), jnp.float32),
        jax.ShapeDtypeStruct((MP, K), jnp.int32),
        jax.ShapeDtypeStruct((MP, K), jnp.int32),
        jax.ShapeDtypeStruct((MP, K * 6), jnp.float32),
        jax.ShapeDtypeStruct((MP, 4), jnp.float32),
    ]
    small = lambda w, d: pl.BlockSpec((R, w), lambda i: (i, 0))
    return pl.pallas_call(
        _topk_body,
        grid=(MP // R,),
        in_specs=[
            pl.BlockSpec((R, 8), lambda i: (i, 0)),
            pl.BlockSpec((8, N), lambda i: (0, 0)),
        ],
        out_specs=[
            small(K, jnp.float32),
            small(K, jnp.float32),
            small(K, jnp.int32),
            small(K, jnp.int32),
            small(K * 6, jnp.float32),
            small(4, jnp.float32),
        ],
        out_shape=out_shapes,
        scratch_shapes=[pltpu.VMEM((R, N), jnp.float32)],
    )(gt_pad, vp)


def _hm_body(hmf_hbm, zero_hbm, pkt_hbm, aux_hbm, bidx_hbm,
             out_hbm, offx_hbm, offy_hbm,
             idx_v, row_v, bidx_v, aux_v, ox_v, oy_v):
    cid = lax.axis_index("c")
    sid = lax.axis_index("s")

    @pl.when((cid == 0) & (sid < NUM_CLASSES))
    def _():
        pltpu.sync_copy(zero_hbm.at[sid], row_v)
        pltpu.sync_copy(hmf_hbm, idx_v)
        ones = jnp.full((16,), 1.0, jnp.float32)
        base = sid * N

        def body(j, carry):
            ids = idx_v[pl.ds(j * 16, 16)]
            msk = (ids >= base) & (ids < base + N)
            loc = jnp.where(msk, ids - base, 0)
            plsc.store_scatter(row_v, [loc], ones, mask=msk)
            return carry

        lax.fori_loop(0, FLAT // 16, body, 0)
        pltpu.sync_copy(row_v, out_hbm.at[sid])

    n_gather = 8
    per_w = FLAT // 16 // n_gather

    @pl.when((cid == 1) & (sid < n_gather))
    def _():
        pltpu.sync_copy(pkt_hbm, row_v)
        pltpu.sync_copy(hmf_hbm, idx_v)
        pltpu.sync_copy(bidx_hbm, bidx_v)
        pltpu.sync_copy(aux_hbm, aux_v)
        base_j = sid * per_w

        def body(j, carry):
            jj = base_j + j
            ids = idx_v[pl.ds(jj * 16, 16)]
            vidx = ids & (N - 1)
            pkv = plsc.load_gather(row_v, [vidx])
            pki = pkv.astype(jnp.int32)
            vx = (pki >> 9).astype(jnp.float32)
            vy = (pki & 511).astype(jnp.float32)
            b4 = bidx_v[pl.ds(jj * 16, 16)] * 4
            cxv = plsc.load_gather(aux_v, [b4])
            cyv = plsc.load_gather(aux_v, [b4 + 1])
            vld = plsc.load_gather(aux_v, [b4 + 2])
            ox_v[pl.ds(j * 16, 16)] = ((cxv - vx) - 0.5) * vld
            oy_v[pl.ds(j * 16, 16)] = ((cyv - vy) - 0.5) * vld
            return carry

        lax.fori_loop(0, per_w, body, 0)
        pltpu.sync_copy(ox_v, offx_hbm.at[pl.ds(base_j * 16, per_w * 16)])
        pltpu.sync_copy(oy_v, offy_hbm.at[pl.ds(base_j * 16, per_w * 16)])


@functools.cache
def _hm_scatter():
    mesh = plsc.VectorSubcoreMesh(core_axis_name="c", subcore_axis_name="s")
    return pl.kernel(
        _hm_body,
        out_type=[
            jax.ShapeDtypeStruct((NUM_CLASSES, N), jnp.float32),
            jax.ShapeDtypeStruct((FLAT,), jnp.float32),
            jax.ShapeDtypeStruct((FLAT,), jnp.float32),
        ],
        mesh=mesh,
        scratch_types=[
            pltpu.VMEM((FLAT,), jnp.int32),
            pltpu.VMEM((N,), jnp.float32),
            pltpu.VMEM((FLAT,), jnp.int32),
            pltpu.VMEM((MP * 4,), jnp.float32),
            pltpu.VMEM((FLAT // 8,), jnp.float32),
            pltpu.VMEM((FLAT // 8,), jnp.float32),
        ],
        compiler_params=pltpu.CompilerParams(needs_layout_passes=False),
    )


def kernel(gt_boxes, spatial_indices):
    vox = spatial_indices.astype(jnp.float32)
    vxh = vox[:, 0] + 0.5
    vyh = vox[:, 1] + 0.5
    pk = vox[:, 0] * 512.0 + vox[:, 1]
    zero_row = jnp.zeros((N,), jnp.float32)
    vp = jnp.stack([vxh, vyh, pk, zero_row, zero_row, zero_row, zero_row,
                    zero_row], axis=0)
    gt_pad = jnp.zeros((MP, 8), jnp.float32).at[:M].set(gt_boxes)

    cd, mask, inds, hmf, rb6, aux = _run_topk(gt_pad, vp)
    bidx = (jnp.arange(FLAT, dtype=jnp.int32) // K).astype(jnp.int32)
    heatmap, offx, offy = _hm_scatter()(
        hmf.reshape(FLAT), jnp.zeros((NUM_CLASSES, N), jnp.float32),
        pk, aux.reshape(MP * 4), bidx)
    off = jnp.stack([offx, offy], axis=-1).reshape(MP, K, 2)
    ret_boxes = jnp.concatenate([off, rb6.reshape(MP, K, 6)], axis=-1)[:M]
    return heatmap, ret_boxes, cd[:M], inds[:M], mask[:M]
